# baseline (speedup 1.0000x reference)
import jax
import jax.numpy as jnp
from jax import lax
from jax.experimental import pallas as pl
from jax.experimental.pallas import tpu as pltpu
from jax.experimental.pallas import tpu_sc as plsc

N_FIELDS = 26
VOCAB = 100000
EMB_DIM = 32
BATCH = 16384
NC, NS = 2, 16
NW = NC * NS
B_PER_W = BATCH // NW
G = 8
NOUT = 1


def _body(idx_hbm, *rest):
    tables = rest[:N_FIELDS]
    out = rest[N_FIELDS]
    idx_v = rest[N_FIELDS + 1]
    slab_v = rest[N_FIELDS + 2]
    outs = rest[N_FIELDS + 3:N_FIELDS + 3 + NOUT]
    isem = rest[N_FIELDS + 3 + NOUT]
    gsem = rest[N_FIELDS + 4 + NOUT]
    wsem = rest[N_FIELDS + 5 + NOUT:N_FIELDS + 5 + 2 * NOUT]

    wid = lax.axis_index("s") * NC + lax.axis_index("c")
    base = wid * B_PER_W

    pend = [
        pltpu.async_copy(
            idx_hbm.at[pl.ds(f * BATCH + base, B_PER_W)],
            idx_v.at[pl.ds(f * B_PER_W, B_PER_W)], isem)
        for f in range(N_FIELDS)
    ]
    for p in pend:
        p.wait()

    for f in range(N_FIELDS):
        ob = outs[0]

        def step(i, _, f=f, ob=ob):
            r0 = i * G
            idx16 = idx_v[pl.ds(f * B_PER_W + r0, 16)]
            rem16 = idx16 & 7
            slab16 = idx16 - rem16
            fetch = []
            for g in range(G):
                row8 = pl.multiple_of(slab16[g], 8)
                fetch.append(pltpu.async_copy(
                    tables[f].at[pl.ds(row8, 8), :], slab_v.at[g], gsem))
            for g in range(G):
                fetch[g].wait()
                rs = rem16[g]
                ob[r0 + g, pl.ds(0, 16)] = slab_v[g, rs, pl.ds(0, 16)]
                ob[r0 + g, pl.ds(16, 16)] = slab_v[g, rs, pl.ds(16, 16)]
            return _

        lax.fori_loop(0, B_PER_W // G, step, None)
        pltpu.async_copy(
            ob, out.at[f, pl.ds(base, B_PER_W), :], wsem[0]).wait()


def kernel(idx_0, idx_1, idx_2, idx_3, idx_4, idx_5, idx_6, idx_7, idx_8, idx_9, idx_10, idx_11, idx_12, idx_13, idx_14, idx_15, idx_16, idx_17, idx_18, idx_19, idx_20, idx_21, idx_22, idx_23, idx_24, idx_25, table_0, table_1, table_2, table_3, table_4, table_5, table_6, table_7, table_8, table_9, table_10, table_11, table_12, table_13, table_14, table_15, table_16, table_17, table_18, table_19, table_20, table_21, table_22, table_23, table_24, table_25):
    idxs = [
        idx_0, idx_1, idx_2, idx_3, idx_4, idx_5, idx_6, idx_7, idx_8, idx_9,
        idx_10, idx_11, idx_12, idx_13, idx_14, idx_15, idx_16, idx_17,
        idx_18, idx_19, idx_20, idx_21, idx_22, idx_23, idx_24, idx_25,
    ]
    tables = [
        table_0, table_1, table_2, table_3, table_4, table_5, table_6,
        table_7, table_8, table_9, table_10, table_11, table_12, table_13,
        table_14, table_15, table_16, table_17, table_18, table_19, table_20,
        table_21, table_22, table_23, table_24, table_25,
    ]
    idx_cat = jnp.concatenate([i.astype(jnp.int32) for i in idxs])

    k = pl.kernel(
        _body,
        out_type=jax.ShapeDtypeStruct((N_FIELDS, BATCH, EMB_DIM), jnp.float32),
        mesh=plsc.VectorSubcoreMesh(
            core_axis_name="c", subcore_axis_name="s",
            num_cores=NC, num_subcores=NS,
        ),
        scratch_types=(
            [pltpu.VMEM((N_FIELDS * B_PER_W + 16,), jnp.int32)]
            + [pltpu.VMEM((G, 8, EMB_DIM), jnp.float32)]
            + [pltpu.VMEM((B_PER_W, EMB_DIM), jnp.float32)] * NOUT
            + [pltpu.SemaphoreType.DMA] * (2 + NOUT)
        ),
    )
    out3 = k(idx_cat, *tables)
    return out3.transpose(1, 0, 2).reshape(BATCH, N_FIELDS * EMB_DIM)


# speedup vs baseline: 2.1815x; 2.1815x over previous
import jax
import jax.numpy as jnp
from jax import lax
from jax.experimental import pallas as pl
from jax.experimental.pallas import tpu as pltpu
from jax.experimental.pallas import tpu_sc as plsc

N_FIELDS = 26
VOCAB = 100000
EMB_DIM = 32
BATCH = 16384
NC, NS = 2, 16
NW = NC * NS
B_PER_W = BATCH // NW
NBUF = 4
LOOK = 2


def _body(idx_hbm, *rest):
    tables = rest[:N_FIELDS]
    out = rest[N_FIELDS]
    idx_bufs = rest[N_FIELDS + 1:N_FIELDS + 1 + NBUF]
    rows = rest[N_FIELDS + 1 + NBUF:N_FIELDS + 1 + 2 * NBUF]
    isem = rest[N_FIELDS + 1 + 2 * NBUF]
    gsem = rest[N_FIELDS + 2 + 2 * NBUF:N_FIELDS + 2 + 3 * NBUF]
    wsem = rest[N_FIELDS + 2 + 3 * NBUF:N_FIELDS + 2 + 4 * NBUF]

    wid = lax.axis_index("s") * NC + lax.axis_index("c")
    base = wid * B_PER_W

    pending_i = {}
    pending_g = {}
    pending_w = {}

    def start_idx(f):
        b = f % NBUF
        pending_i[f] = pltpu.async_copy(
            idx_hbm.at[pl.ds(f * BATCH + base, B_PER_W)], idx_bufs[b], isem)

    def start_gather(f):
        b = f % NBUF
        pending_i.pop(f).wait()
        pending_g[f] = pltpu.async_copy(
            tables[f].at[idx_bufs[b]], rows[b], gsem[b])

    for f in range(LOOK):
        start_idx(f)
    for f in range(LOOK):
        start_gather(f)

    for f in range(N_FIELDS):
        b = f % NBUF
        g = f + LOOK
        if g < N_FIELDS:
            start_idx(g)
        pending_g.pop(f).wait()
        pending_w[f] = pltpu.async_copy(
            rows[b],
            out.at[pl.ds(base, B_PER_W), pl.ds(f * EMB_DIM, EMB_DIM)],
            wsem[b])
        if g < N_FIELDS:
            if g >= NBUF:
                pending_w.pop(g - NBUF).wait()
            start_gather(g)

    for f in sorted(pending_w):
        pending_w.pop(f).wait()


def kernel(idx_0, idx_1, idx_2, idx_3, idx_4, idx_5, idx_6, idx_7, idx_8, idx_9, idx_10, idx_11, idx_12, idx_13, idx_14, idx_15, idx_16, idx_17, idx_18, idx_19, idx_20, idx_21, idx_22, idx_23, idx_24, idx_25, table_0, table_1, table_2, table_3, table_4, table_5, table_6, table_7, table_8, table_9, table_10, table_11, table_12, table_13, table_14, table_15, table_16, table_17, table_18, table_19, table_20, table_21, table_22, table_23, table_24, table_25):
    idxs = [
        idx_0, idx_1, idx_2, idx_3, idx_4, idx_5, idx_6, idx_7, idx_8, idx_9,
        idx_10, idx_11, idx_12, idx_13, idx_14, idx_15, idx_16, idx_17,
        idx_18, idx_19, idx_20, idx_21, idx_22, idx_23, idx_24, idx_25,
    ]
    tables = [
        table_0, table_1, table_2, table_3, table_4, table_5, table_6,
        table_7, table_8, table_9, table_10, table_11, table_12, table_13,
        table_14, table_15, table_16, table_17, table_18, table_19, table_20,
        table_21, table_22, table_23, table_24, table_25,
    ]
    idx_cat = jnp.concatenate([i.astype(jnp.int32) for i in idxs])

    k = pl.kernel(
        _body,
        out_type=jax.ShapeDtypeStruct((BATCH, N_FIELDS * EMB_DIM), jnp.float32),
        mesh=plsc.VectorSubcoreMesh(
            core_axis_name="c", subcore_axis_name="s",
            num_cores=NC, num_subcores=NS,
        ),
        scratch_types=(
            [pltpu.VMEM((B_PER_W,), jnp.int32)] * NBUF
            + [pltpu.VMEM((B_PER_W, EMB_DIM), jnp.float32)] * NBUF
            + [pltpu.SemaphoreType.DMA] * (1 + 2 * NBUF)
        ),
        compiler_params=pltpu.CompilerParams(use_tc_tiling_on_sc=False),
    )
    return k(idx_cat, *tables)
